# baseline (speedup 1.0000x reference)
import functools

import jax
import jax.numpy as jnp
from jax import lax
from jax.experimental import pallas as pl
from jax.experimental.pallas import tpu as pltpu
from jax.experimental.pallas import tpu_sc as plsc

NC = 2
NS = 16
NW = NC * NS
LANES = 16


@functools.lru_cache(maxsize=None)
def _build(B, Lp, D):
    RW = B // NW
    CK = RW // 2
    IPC = 2 * Lp
    NBUF = 2
    G = CK // NBUF
    NV = D // LANES

    mesh = plsc.VectorSubcoreMesh(core_axis_name="c", subcore_axis_name="s")

    @functools.partial(
        pl.kernel,
        out_type=jax.ShapeDtypeStruct((B, D), jnp.float32),
        mesh=mesh,
        compiler_params=pltpu.CompilerParams(use_tc_tiling_on_sc=False),
        scratch_types=[
            pltpu.VMEM((RW * Lp,), jnp.int32),
            pltpu.VMEM((RW * Lp + LANES,), jnp.float32),
            pltpu.VMEM((NBUF, IPC, D), jnp.float32),
            pltpu.VMEM((RW, D), jnp.float32),
            pltpu.SemaphoreType.DMA,
            pltpu.SemaphoreType.DMA,
        ],
    )
    def k(x_hbm, w_hbm, table_hbm, out_hbm, xidx_v, w_v, emb_v, out_v, sem0, sem1):
        sems = (sem0, sem1)
        wid = lax.axis_index("s") * NC + lax.axis_index("c")
        base = wid * RW

        pltpu.sync_copy(x_hbm.at[pl.ds(base * Lp, RW * Lp)], xidx_v)
        pltpu.sync_copy(w_hbm.at[pl.ds(base * Lp, RW * Lp)], w_v.at[pl.ds(0, RW * Lp)])

        def gather(c, b):
            return pltpu.make_async_copy(
                table_hbm.at[xidx_v.at[pl.ds(c * IPC, IPC)]],
                emb_v.at[b],
                sems[b],
            )

        for b in range(NBUF):
            gather(b, b).start()

        def compute(c, b):
            for rr in range(2):
                r = c * 2 + rr
                wrow = [
                    w_v[pl.ds(r * Lp + q * LANES, LANES)]
                    for q in range((Lp + LANES - 1) // LANES)
                ]
                accs = [jnp.zeros((LANES,), jnp.float32) for _ in range(NV)]
                for l in range(Lp):
                    wl = jnp.full((LANES,), wrow[l // LANES][l % LANES])
                    row = rr * Lp + l
                    for j in range(NV):
                        accs[j] = accs[j] + wl * emb_v[b, row, pl.ds(j * LANES, LANES)]
                for j in range(NV):
                    out_v[r, pl.ds(j * LANES, LANES)] = accs[j]

        def group(g, carry):
            for b in range(NBUF):
                c = g * NBUF + b
                gather(c, b).wait()
                compute(c, b)
                nc = c + NBUF

                @pl.when(nc < CK)
                def _():
                    gather(nc, b).start()

            return carry

        lax.fori_loop(0, G, group, 0)

        pltpu.sync_copy(out_v, out_hbm.at[pl.ds(base, RW)])

    return k


def kernel(x, w, weight):
    B, L = x.shape
    _, D = weight.shape
    Lp = 52
    pad = Lp - L
    x_p = jnp.pad(x.astype(jnp.int32), ((0, 0), (0, pad))).reshape(-1)
    w_p = jnp.pad(w, ((0, 0), (0, pad))).reshape(-1)
    return _build(B, Lp, D)(x_p, w_p, weight)


# speedup vs baseline: 1.2313x; 1.2313x over previous
import functools

import jax
import jax.numpy as jnp
from jax import lax
from jax.experimental import pallas as pl
from jax.experimental.pallas import tpu as pltpu
from jax.experimental.pallas import tpu_sc as plsc

NC = 2
NS = 16
NW = NC * NS
LANES = 16


@functools.lru_cache(maxsize=None)
def _build(B, L, D):
    RW = B // NW
    NBUF = 4
    G = RW // NBUF
    NV = D // LANES
    NQ = (L + LANES - 1) // LANES

    mesh = plsc.VectorSubcoreMesh(core_axis_name="c", subcore_axis_name="s")

    @functools.partial(
        pl.kernel,
        out_type=jax.ShapeDtypeStruct((B, D), jnp.float32),
        mesh=mesh,
        compiler_params=pltpu.CompilerParams(use_tc_tiling_on_sc=False),
        scratch_types=[
            pltpu.VMEM((RW, L), jnp.int32),
            pltpu.VMEM((RW, L), jnp.float32),
            pltpu.VMEM((NBUF, L, D), jnp.float32),
            pltpu.VMEM((RW, D), jnp.float32),
            pltpu.SemaphoreType.DMA,
            pltpu.SemaphoreType.DMA,
            pltpu.SemaphoreType.DMA,
            pltpu.SemaphoreType.DMA,
        ],
    )
    def k(x_hbm, w_hbm, table_hbm, out_hbm, xidx_v, w_v, emb_v, out_v, *sems):
        wid = lax.axis_index("s") * NC + lax.axis_index("c")
        base = wid * RW

        pltpu.sync_copy(x_hbm.at[pl.ds(base, RW)], xidx_v)
        pltpu.sync_copy(w_hbm.at[pl.ds(base, RW)], w_v)

        offs = [min(q * LANES, L - LANES) for q in range(NQ)]

        def gather(r, b):
            return pltpu.make_async_copy(
                table_hbm.at[xidx_v.at[r]], emb_v.at[b], sems[b]
            )

        for b in range(NBUF):
            gather(b, b).start()

        def compute(r, b):
            wrow = [w_v[r, pl.ds(off, LANES)] for off in offs]
            accs = [jnp.zeros((LANES,), jnp.float32) for _ in range(NV)]
            for l in range(L):
                q = min(l // LANES, NQ - 1)
                wl = jnp.full((LANES,), wrow[q][l - offs[q]])
                for j in range(NV):
                    accs[j] = accs[j] + wl * emb_v[b, l, pl.ds(j * LANES, LANES)]
            for j in range(NV):
                out_v[r, pl.ds(j * LANES, LANES)] = accs[j]

        def group(g, carry):
            for b in range(NBUF):
                r = g * NBUF + b
                gather(r, b).wait()
                compute(r, b)
                nr = r + NBUF

                @pl.when(nr < RW)
                def _():
                    gather(nr, b).start()

            return carry

        lax.fori_loop(0, G, group, 0)

        pltpu.sync_copy(out_v, out_hbm.at[pl.ds(base, RW)])

    return k


def kernel(x, w, weight):
    B, L = x.shape
    _, D = weight.shape
    return _build(B, L, D)(x.astype(jnp.int32), w, weight)


# speedup vs baseline: 1.5797x; 1.2830x over previous
import functools

import jax
import jax.numpy as jnp
from jax import lax
from jax.experimental import pallas as pl
from jax.experimental.pallas import tpu as pltpu
from jax.experimental.pallas import tpu_sc as plsc

NC = 2
NS = 16
NW = NC * NS
LANES = 16
FB = 2048


@functools.lru_cache(maxsize=None)
def _build_transpose(C, F):
    G = -(-F // FB)

    def body(in_ref, out_ref):
        t = in_ref[...]
        tt = t.T
        out_ref[...] = jnp.concatenate([tt[: FB // 2], tt[FB // 2 :]], axis=1)

    return pl.pallas_call(
        body,
        grid=(G,),
        in_specs=[pl.BlockSpec((C, FB), lambda i: (0, i))],
        out_specs=pl.BlockSpec((FB // 2, 2 * C), lambda i: (i, 0)),
        out_shape=jax.ShapeDtypeStruct((G * FB // 2, 2 * C), jnp.float32),
    )


@functools.lru_cache(maxsize=None)
def _build(B, L, D, NT):
    RW = B // NW
    NBUF = 4
    NV = D // LANES
    NQ = (L + LANES - 1) // LANES

    mesh = plsc.VectorSubcoreMesh(core_axis_name="c", subcore_axis_name="s")

    @functools.partial(
        pl.kernel,
        out_type=jax.ShapeDtypeStruct((B, D), jnp.float32),
        mesh=mesh,
        compiler_params=pltpu.CompilerParams(use_tc_tiling_on_sc=False),
        scratch_types=[
            pltpu.VMEM((RW, L), jnp.int32),
            pltpu.VMEM((RW, L), jnp.int32),
            pltpu.VMEM((RW, L), jnp.float32),
            pltpu.VMEM((NBUF, L, D), jnp.float32),
            pltpu.VMEM((RW, D), jnp.float32),
            pltpu.SemaphoreType.DMA,
            pltpu.SemaphoreType.DMA,
            pltpu.SemaphoreType.DMA,
            pltpu.SemaphoreType.DMA,
        ],
    )
    def k(x_hbm, w_hbm, table_hbm, out_hbm, xidx_v, xmap_v, w_v, emb_v, out_v, *sems):
        wid = lax.axis_index("s") * NC + lax.axis_index("c")
        base = wid * RW

        pltpu.sync_copy(x_hbm.at[pl.ds(base, RW)], xidx_v)
        pltpu.sync_copy(w_hbm.at[pl.ds(base, RW)], w_v)

        offs = [min(q * LANES, L - LANES) for q in range(NQ)]

        def remap(r):
            for off in offs:
                f = xidx_v[r, pl.ds(off, LANES)]
                p = (f & -FB) | ((f & (FB // 2 - 1)) << 1) | ((f >> 10) & 1)
                xmap_v[r, pl.ds(off, LANES)] = p

        def gather(r, b):
            return pltpu.make_async_copy(
                table_hbm.at[xmap_v.at[r]], emb_v.at[b], sems[b]
            )

        for b in range(NBUF):
            remap(b)
            gather(b, b).start()

        def compute(r, b):
            wrow = [w_v[r, pl.ds(off, LANES)] for off in offs]
            accs = [jnp.zeros((LANES,), jnp.float32) for _ in range(NV)]
            for l in range(L):
                q = min(l // LANES, NQ - 1)
                wl = jnp.full((LANES,), wrow[q][l - offs[q]])
                for j in range(NV):
                    accs[j] = accs[j] + wl * emb_v[b, l, pl.ds(j * LANES, LANES)]
            for j in range(NV):
                out_v[r, pl.ds(j * LANES, LANES)] = accs[j]

        def group(g, carry):
            for b in range(NBUF):
                r = g * NBUF + b
                gather(r, b).wait()
                compute(r, b)
                nr = r + NBUF

                @pl.when(nr < RW)
                def _():
                    remap(nr)
                    gather(nr, b).start()

            return carry

        lax.fori_loop(0, RW // NBUF, group, 0)

        pltpu.sync_copy(out_v, out_hbm.at[pl.ds(base, RW)])

    return k


def kernel(x, w, weight):
    B, L = x.shape
    N, D = weight.shape
    w2 = _build_transpose(D, N)(weight.T)
    NT = 2 * w2.shape[0]
    table = w2.reshape(NT, D)
    return _build(B, L, D, NT)(x.astype(jnp.int32), w, table)


# speedup vs baseline: 2.1674x; 1.3720x over previous
import functools

import jax
import jax.numpy as jnp
from jax import lax
from jax.experimental import pallas as pl
from jax.experimental.pallas import tpu as pltpu
from jax.experimental.pallas import tpu_sc as plsc

NC = 2
NS = 16
NW = NC * NS
LANES = 16
FB = 4096


@functools.lru_cache(maxsize=None)
def _build_transpose(C, F):
    G = -(-F // FB)

    def body(in_ref, out_ref):
        t = in_ref[...]
        eye = jnp.eye(C, dtype=jnp.float32)
        e_lo = jnp.concatenate([eye, jnp.zeros((C, C), jnp.float32)], axis=1)
        e_hi = jnp.concatenate([jnp.zeros((C, C), jnp.float32), eye], axis=1)
        dn = (((0,), (0,)), ((), ()))
        lo = lax.dot_general(t[:, : FB // 2], e_lo, dn,
                             preferred_element_type=jnp.float32)
        hi = lax.dot_general(t[:, FB // 2 :], e_hi, dn,
                             preferred_element_type=jnp.float32)
        out_ref[...] = lo + hi

    return pl.pallas_call(
        body,
        grid=(G,),
        in_specs=[pl.BlockSpec((C, FB), lambda i: (0, i))],
        out_specs=pl.BlockSpec((FB // 2, 2 * C), lambda i: (i, 0)),
        out_shape=jax.ShapeDtypeStruct((G * FB // 2, 2 * C), jnp.float32),
    )


@functools.lru_cache(maxsize=None)
def _build(B, L, D, NT):
    RW = B // NW
    NBUF = 4
    NV = D // LANES
    NQ = (L + LANES - 1) // LANES

    mesh = plsc.VectorSubcoreMesh(core_axis_name="c", subcore_axis_name="s")

    @functools.partial(
        pl.kernel,
        out_type=jax.ShapeDtypeStruct((B, D), jnp.float32),
        mesh=mesh,
        compiler_params=pltpu.CompilerParams(use_tc_tiling_on_sc=False),
        scratch_types=[
            pltpu.VMEM((RW, L), jnp.int32),
            pltpu.VMEM((RW, L), jnp.int32),
            pltpu.VMEM((RW, L), jnp.float32),
            pltpu.VMEM((NBUF, L, D), jnp.float32),
            pltpu.VMEM((RW, D), jnp.float32),
            pltpu.SemaphoreType.DMA,
            pltpu.SemaphoreType.DMA,
            pltpu.SemaphoreType.DMA,
            pltpu.SemaphoreType.DMA,
        ],
    )
    def k(x_hbm, w_hbm, table_hbm, out_hbm, xidx_v, xmap_v, w_v, emb_v, out_v, *sems):
        wid = lax.axis_index("s") * NC + lax.axis_index("c")
        base = wid * RW

        pltpu.sync_copy(x_hbm.at[pl.ds(base, RW)], xidx_v)
        pltpu.sync_copy(w_hbm.at[pl.ds(base, RW)], w_v)

        offs = [min(q * LANES, L - LANES) for q in range(NQ)]

        def remap(r):
            for off in offs:
                f = xidx_v[r, pl.ds(off, LANES)]
                sh = (FB // 2).bit_length() - 1
                p = (f & -FB) | ((f & (FB // 2 - 1)) << 1) | ((f >> sh) & 1)
                xmap_v[r, pl.ds(off, LANES)] = p

        def gather(r, b):
            return pltpu.make_async_copy(
                table_hbm.at[xmap_v.at[r]], emb_v.at[b], sems[b]
            )

        for b in range(NBUF):
            remap(b)
            gather(b, b).start()

        def compute(r, b):
            wrow = [w_v[r, pl.ds(off, LANES)] for off in offs]
            accs = [jnp.zeros((LANES,), jnp.float32) for _ in range(NV)]
            for l in range(L):
                q = min(l // LANES, NQ - 1)
                wl = jnp.full((LANES,), wrow[q][l - offs[q]])
                for j in range(NV):
                    accs[j] = accs[j] + wl * emb_v[b, l, pl.ds(j * LANES, LANES)]
            for j in range(NV):
                out_v[r, pl.ds(j * LANES, LANES)] = accs[j]

        def group(g, carry):
            for b in range(NBUF):
                r = g * NBUF + b
                gather(r, b).wait()
                compute(r, b)
                nr = r + NBUF

                @pl.when(nr < RW)
                def _():
                    remap(nr)
                    gather(nr, b).start()

            return carry

        lax.fori_loop(0, RW // NBUF, group, 0)

        pltpu.sync_copy(out_v, out_hbm.at[pl.ds(base, RW)])

    return k


def kernel(x, w, weight):
    B, L = x.shape
    N, D = weight.shape
    w2 = _build_transpose(D, N)(weight.T)
    NT = 2 * w2.shape[0]
    table = w2.reshape(NT, D)
    return _build(B, L, D, NT)(x.astype(jnp.int32), w, table)


# speedup vs baseline: 2.7354x; 1.2620x over previous
import functools

import jax
import jax.numpy as jnp
from jax import lax
from jax.experimental import pallas as pl
from jax.experimental.pallas import tpu as pltpu
from jax.experimental.pallas import tpu_sc as plsc

NC = 2
NS = 16
NW = NC * NS
LANES = 16
FB = 8192


@functools.lru_cache(maxsize=None)
def _build_transpose(C, F):
    G = -(-F // FB)

    def body(in_ref, out_ref):
        t = in_ref[...]
        eye = jnp.eye(C, dtype=jnp.float32)
        e_lo = jnp.concatenate([eye, jnp.zeros((C, C), jnp.float32)], axis=1)
        e_hi = jnp.concatenate([jnp.zeros((C, C), jnp.float32), eye], axis=1)
        dn = (((0,), (0,)), ((), ()))
        lo = lax.dot_general(t[:, : FB // 2], e_lo, dn,
                             preferred_element_type=jnp.float32)
        hi = lax.dot_general(t[:, FB // 2 :], e_hi, dn,
                             preferred_element_type=jnp.float32)
        out_ref[...] = lo + hi

    return pl.pallas_call(
        body,
        grid=(G,),
        in_specs=[pl.BlockSpec((C, FB), lambda i: (0, i))],
        out_specs=pl.BlockSpec((FB // 2, 2 * C), lambda i: (i, 0)),
        out_shape=jax.ShapeDtypeStruct((G * FB // 2, 2 * C), jnp.float32),
    )


@functools.lru_cache(maxsize=None)
def _build(B, L, D, NT):
    RW = B // NW
    NBUF = 4
    NV = D // LANES
    NQ = (L + LANES - 1) // LANES

    mesh = plsc.VectorSubcoreMesh(core_axis_name="c", subcore_axis_name="s")

    @functools.partial(
        pl.kernel,
        out_type=jax.ShapeDtypeStruct((B, D), jnp.float32),
        mesh=mesh,
        compiler_params=pltpu.CompilerParams(use_tc_tiling_on_sc=False),
        scratch_types=[
            pltpu.VMEM((RW, L), jnp.int32),
            pltpu.VMEM((RW, L), jnp.int32),
            pltpu.VMEM((RW, L), jnp.float32),
            pltpu.VMEM((NBUF, L, D), jnp.float32),
            pltpu.VMEM((RW, D), jnp.float32),
            pltpu.SemaphoreType.DMA,
            pltpu.SemaphoreType.DMA,
            pltpu.SemaphoreType.DMA,
            pltpu.SemaphoreType.DMA,
        ],
    )
    def k(x_hbm, w_hbm, table_hbm, out_hbm, xidx_v, xmap_v, w_v, emb_v, out_v, *sems):
        wid = lax.axis_index("s") * NC + lax.axis_index("c")
        base = wid * RW

        pltpu.sync_copy(x_hbm.at[pl.ds(base, RW)], xidx_v)
        pltpu.sync_copy(w_hbm.at[pl.ds(base, RW)], w_v)

        offs = [min(q * LANES, L - LANES) for q in range(NQ)]

        def remap(r):
            for off in offs:
                f = xidx_v[r, pl.ds(off, LANES)]
                sh = (FB // 2).bit_length() - 1
                p = (f & -FB) | ((f & (FB // 2 - 1)) << 1) | ((f >> sh) & 1)
                xmap_v[r, pl.ds(off, LANES)] = p

        def gather(r, b):
            return pltpu.make_async_copy(
                table_hbm.at[xmap_v.at[r]], emb_v.at[b], sems[b]
            )

        for b in range(NBUF):
            remap(b)
            gather(b, b).start()

        def compute(r, b):
            wrow = [w_v[r, pl.ds(off, LANES)] for off in offs]
            accs = [jnp.zeros((LANES,), jnp.float32) for _ in range(NV)]
            for l in range(L):
                q = min(l // LANES, NQ - 1)
                wl = jnp.full((LANES,), wrow[q][l - offs[q]])
                for j in range(NV):
                    accs[j] = accs[j] + wl * emb_v[b, l, pl.ds(j * LANES, LANES)]
            for j in range(NV):
                out_v[r, pl.ds(j * LANES, LANES)] = accs[j]

        def group(g, carry):
            for b in range(NBUF):
                r = g * NBUF + b
                gather(r, b).wait()
                compute(r, b)
                nr = r + NBUF

                @pl.when(nr < RW)
                def _():
                    remap(nr)
                    gather(nr, b).start()

            return carry

        lax.fori_loop(0, RW // NBUF, group, 0)

        pltpu.sync_copy(out_v, out_hbm.at[pl.ds(base, RW)])

    return k


def kernel(x, w, weight):
    B, L = x.shape
    N, D = weight.shape
    w2 = _build_transpose(D, N)(weight.T)
    NT = 2 * w2.shape[0]
    table = w2.reshape(NT, D)
    return _build(B, L, D, NT)(x.astype(jnp.int32), w, table)


# speedup vs baseline: 3.1106x; 1.1372x over previous
import functools

import jax
import jax.numpy as jnp
from jax import lax
from jax.experimental import pallas as pl
from jax.experimental.pallas import tpu as pltpu
from jax.experimental.pallas import tpu_sc as plsc

NC = 2
NS = 16
NW = NC * NS
LANES = 16
FB = 16384


@functools.lru_cache(maxsize=None)
def _build_transpose(C, F):
    G = -(-F // FB)

    def body(in_ref, out_ref):
        t = in_ref[...]
        eye = jnp.eye(C, dtype=jnp.float32)
        e_lo = jnp.concatenate([eye, jnp.zeros((C, C), jnp.float32)], axis=1)
        e_hi = jnp.concatenate([jnp.zeros((C, C), jnp.float32), eye], axis=1)
        dn = (((0,), (0,)), ((), ()))
        lo = lax.dot_general(t[:, : FB // 2], e_lo, dn,
                             preferred_element_type=jnp.float32)
        hi = lax.dot_general(t[:, FB // 2 :], e_hi, dn,
                             preferred_element_type=jnp.float32)
        out_ref[...] = lo + hi

    return pl.pallas_call(
        body,
        grid=(G,),
        in_specs=[pl.BlockSpec((C, FB), lambda i: (0, i))],
        out_specs=pl.BlockSpec((FB // 2, 2 * C), lambda i: (i, 0)),
        out_shape=jax.ShapeDtypeStruct((G * FB // 2, 2 * C), jnp.float32),
    )


@functools.lru_cache(maxsize=None)
def _build(B, L, D, NT):
    RW = B // NW
    NBUF = 4
    NV = D // LANES
    NQ = (L + LANES - 1) // LANES

    mesh = plsc.VectorSubcoreMesh(core_axis_name="c", subcore_axis_name="s")

    @functools.partial(
        pl.kernel,
        out_type=jax.ShapeDtypeStruct((B, D), jnp.float32),
        mesh=mesh,
        compiler_params=pltpu.CompilerParams(use_tc_tiling_on_sc=False),
        scratch_types=[
            pltpu.VMEM((RW, L), jnp.int32),
            pltpu.VMEM((RW, L), jnp.int32),
            pltpu.VMEM((RW, L), jnp.float32),
            pltpu.VMEM((NBUF, L, D), jnp.float32),
            pltpu.VMEM((RW, D), jnp.float32),
            pltpu.SemaphoreType.DMA,
            pltpu.SemaphoreType.DMA,
            pltpu.SemaphoreType.DMA,
            pltpu.SemaphoreType.DMA,
        ],
    )
    def k(x_hbm, w_hbm, table_hbm, out_hbm, xidx_v, xmap_v, w_v, emb_v, out_v, *sems):
        wid = lax.axis_index("s") * NC + lax.axis_index("c")
        base = wid * RW

        pltpu.sync_copy(x_hbm.at[pl.ds(base, RW)], xidx_v)
        pltpu.sync_copy(w_hbm.at[pl.ds(base, RW)], w_v)

        offs = [min(q * LANES, L - LANES) for q in range(NQ)]

        def remap(r):
            for off in offs:
                f = xidx_v[r, pl.ds(off, LANES)]
                sh = (FB // 2).bit_length() - 1
                p = (f & -FB) | ((f & (FB // 2 - 1)) << 1) | ((f >> sh) & 1)
                xmap_v[r, pl.ds(off, LANES)] = p

        def gather(r, b):
            return pltpu.make_async_copy(
                table_hbm.at[xmap_v.at[r]], emb_v.at[b], sems[b]
            )

        for b in range(NBUF):
            remap(b)
            gather(b, b).start()

        def compute(r, b):
            wrow = [w_v[r, pl.ds(off, LANES)] for off in offs]
            accs = [jnp.zeros((LANES,), jnp.float32) for _ in range(NV)]
            for l in range(L):
                q = min(l // LANES, NQ - 1)
                wl = jnp.full((LANES,), wrow[q][l - offs[q]])
                for j in range(NV):
                    accs[j] = accs[j] + wl * emb_v[b, l, pl.ds(j * LANES, LANES)]
            for j in range(NV):
                out_v[r, pl.ds(j * LANES, LANES)] = accs[j]

        def group(g, carry):
            for b in range(NBUF):
                r = g * NBUF + b
                gather(r, b).wait()
                compute(r, b)
                nr = r + NBUF

                @pl.when(nr < RW)
                def _():
                    remap(nr)
                    gather(nr, b).start()

            return carry

        lax.fori_loop(0, RW // NBUF, group, 0)

        pltpu.sync_copy(out_v, out_hbm.at[pl.ds(base, RW)])

    return k


def kernel(x, w, weight):
    B, L = x.shape
    N, D = weight.shape
    w2 = _build_transpose(D, N)(weight.T)
    NT = 2 * w2.shape[0]
    table = w2.reshape(NT, D)
    return _build(B, L, D, NT)(x.astype(jnp.int32), w, table)


# speedup vs baseline: 3.3849x; 1.0882x over previous
import functools

import jax
import jax.numpy as jnp
from jax import lax
from jax.experimental import pallas as pl
from jax.experimental.pallas import tpu as pltpu
from jax.experimental.pallas import tpu_sc as plsc

NC = 2
NS = 16
NW = NC * NS
LANES = 16
FB = 32768


@functools.lru_cache(maxsize=None)
def _build_transpose(C, F):
    G = -(-F // FB)

    def body(in_ref, out_ref):
        t = in_ref[...]
        eye = jnp.eye(C, dtype=jnp.float32)
        zero = jnp.zeros((C, C), jnp.float32)
        e_lo = jnp.concatenate([eye, zero], axis=1)
        e_hi = jnp.concatenate([zero, eye], axis=1)
        dn = (((0,), (0,)), ((), ()))
        lo = lax.dot_general(t[:, : FB // 2], e_lo, dn,
                             preferred_element_type=jnp.float32)
        hi = lax.dot_general(t[:, FB // 2 :], e_hi, dn,
                             preferred_element_type=jnp.float32)
        out_ref[...] = lo + hi

    return pl.pallas_call(
        body,
        grid=(G,),
        in_specs=[pl.BlockSpec((C, FB), lambda i: (0, i))],
        out_specs=pl.BlockSpec((FB // 2, 2 * C), lambda i: (i, 0)),
        out_shape=jax.ShapeDtypeStruct((G * FB // 2, 2 * C), jnp.float32),
    )


@functools.lru_cache(maxsize=None)
def _build(B, L, D, NT):
    RW = B // NW
    NBUF = 4
    NV = D // LANES
    NQ = (L + LANES - 1) // LANES

    mesh = plsc.VectorSubcoreMesh(core_axis_name="c", subcore_axis_name="s")

    @functools.partial(
        pl.kernel,
        out_type=jax.ShapeDtypeStruct((B, D), jnp.float32),
        mesh=mesh,
        compiler_params=pltpu.CompilerParams(use_tc_tiling_on_sc=False),
        scratch_types=[
            pltpu.VMEM((RW, L), jnp.int32),
            pltpu.VMEM((RW, L), jnp.int32),
            pltpu.VMEM((RW, L), jnp.float32),
            pltpu.VMEM((NBUF, L, D), jnp.float32),
            pltpu.VMEM((RW, D), jnp.float32),
            pltpu.SemaphoreType.DMA,
            pltpu.SemaphoreType.DMA,
            pltpu.SemaphoreType.DMA,
            pltpu.SemaphoreType.DMA,
        ],
    )
    def k(x_hbm, w_hbm, table_hbm, out_hbm, xidx_v, xmap_v, w_v, emb_v, out_v, *sems):
        wid = lax.axis_index("s") * NC + lax.axis_index("c")
        base = wid * RW

        pltpu.sync_copy(x_hbm.at[pl.ds(base, RW)], xidx_v)
        pltpu.sync_copy(w_hbm.at[pl.ds(base, RW)], w_v)

        offs = [min(q * LANES, L - LANES) for q in range(NQ)]

        def remap(r):
            sh = (FB // 2).bit_length() - 1
            for off in offs:
                f = xidx_v[r, pl.ds(off, LANES)]
                p = (f & -FB) | ((f & (FB // 2 - 1)) << 1) | ((f >> sh) & 1)
                xmap_v[r, pl.ds(off, LANES)] = p

        def gather(r, b):
            return pltpu.make_async_copy(
                table_hbm.at[xmap_v.at[r]], emb_v.at[b], sems[b]
            )

        for b in range(NBUF):
            remap(b)
            gather(b, b).start()

        def compute(r, b):
            wrow = [w_v[r, pl.ds(off, LANES)] for off in offs]
            accs = [jnp.zeros((LANES,), jnp.float32) for _ in range(NV)]
            for l in range(L):
                q = min(l // LANES, NQ - 1)
                wl = jnp.full((LANES,), wrow[q][l - offs[q]])
                for j in range(NV):
                    accs[j] = accs[j] + wl * emb_v[b, l, pl.ds(j * LANES, LANES)]
            for j in range(NV):
                out_v[r, pl.ds(j * LANES, LANES)] = accs[j]

        def group(g, carry):
            for b in range(NBUF):
                r = g * NBUF + b
                gather(r, b).wait()
                compute(r, b)
                nr = r + NBUF

                @pl.when(nr < RW)
                def _():
                    remap(nr)
                    gather(nr, b).start()

            return carry

        lax.fori_loop(0, RW // NBUF, group, 0)

        pltpu.sync_copy(out_v, out_hbm.at[pl.ds(base, RW)])

    return k


def kernel(x, w, weight):
    B, L = x.shape
    N, D = weight.shape
    w2 = _build_transpose(D, N)(weight.T)
    NT = 2 * w2.shape[0]
    table = w2.reshape(NT, D)
    return _build(B, L, D, NT)(x.astype(jnp.int32), w, table)
